# Initial kernel scaffold; baseline (speedup 1.0000x reference)
import jax
import jax.numpy as jnp
from jax.experimental import pallas as pl


def kernel(x, edge_index, batch, W, b, ln_w, ln_b):
    raise NotImplementedError("write your pallas kernel here")


# speedup vs baseline: 6.6154x; 6.6154x over previous
import functools

import jax
import jax.numpy as jnp
from jax import lax
from jax.experimental import pallas as pl
from jax.experimental.pallas import tpu as pltpu
from jax.experimental.pallas import tpu_sc as plsc

N_NODES = 10000
D_FEAT = 128
LN_EPS = 1e-5

NC = 2
NS = 16
CH = 128
ZROWS = 16
NPAD = 10240


def _sc_aggregate(x, src, dst):
    e = src.shape[0]
    epc = e // NC
    epw = epc // NS
    nfull = epw // CH
    tail = epw - nfull * CH
    rows_per_sub = NPAD // NS

    mesh = plsc.VectorSubcoreMesh(core_axis_name="c", subcore_axis_name="s")

    @functools.partial(
        pl.kernel,
        out_type=jax.ShapeDtypeStruct((NC, NPAD, D_FEAT), jnp.float32),
        mesh=mesh,
        scratch_types=[
            pltpu.VMEM_SHARED((NPAD, D_FEAT), jnp.float32),
            pltpu.VMEM((CH,), jnp.int32),
            pltpu.VMEM((CH,), jnp.int32),
            pltpu.VMEM((CH, D_FEAT), jnp.float32),
            pltpu.VMEM((tail if tail else 8,), jnp.int32),
            pltpu.VMEM((tail if tail else 8,), jnp.int32),
            pltpu.VMEM((ZROWS, D_FEAT), jnp.float32),
            pltpu.SemaphoreType.DMA,
        ],
    )
    def k(x_hbm, src_hbm, dst_hbm, out_hbm,
          aggr_sh, src_v, dst_v, rows_v, src_t, dst_t, zbuf, sem):
        c = lax.axis_index("c")
        s = lax.axis_index("s")

        @pl.loop(0, ZROWS)
        def _(i):
            @pl.loop(0, D_FEAT, step=16)
            def _(j):
                zbuf[i, pl.ds(j, 16)] = jnp.zeros((16,), jnp.float32)

        @pl.loop(0, rows_per_sub // ZROWS)
        def _(i):
            pltpu.sync_copy(
                zbuf, aggr_sh.at[pl.ds(s * rows_per_sub + i * ZROWS, ZROWS)])

        plsc.subcore_barrier()

        base = (c * NS + s) * epw

        @pl.loop(0, nfull)
        def _(i):
            off = base + i * CH
            pltpu.sync_copy(src_hbm.at[pl.ds(off, CH)], src_v)
            pltpu.sync_copy(dst_hbm.at[pl.ds(off, CH)], dst_v)
            pltpu.async_copy(x_hbm.at[src_v], rows_v, sem).wait()
            pltpu.sync_copy(rows_v, aggr_sh.at[dst_v], add=True)

        if tail:
            toff = base + nfull * CH
            pltpu.sync_copy(src_hbm.at[pl.ds(toff, tail)], src_t)
            pltpu.sync_copy(dst_hbm.at[pl.ds(toff, tail)], dst_t)
            pltpu.async_copy(x_hbm.at[src_t], rows_v.at[pl.ds(0, tail)],
                             sem).wait()
            pltpu.sync_copy(rows_v.at[pl.ds(0, tail)], aggr_sh.at[dst_t],
                            add=True)

        plsc.subcore_barrier()

        pltpu.sync_copy(
            aggr_sh.at[pl.ds(s * rows_per_sub, rows_per_sub)],
            out_hbm.at[c, pl.ds(s * rows_per_sub, rows_per_sub)])

    return k(x, src, dst)


def _tc_update(x, partials, w, b, ln_w, ln_b):
    blk = 1000
    grid = (N_NODES // blk,)
    partials = lax.slice(partials, (0, 0, 0), (NC, N_NODES, D_FEAT))

    def body(x_ref, p_ref, w_ref, b_ref, lnw_ref, lnb_ref, o_ref):
        h = x_ref[...] + p_ref[0] + p_ref[1]
        y = lax.dot_general(h, w_ref[...], (((1,), (1,)), ((), ())),
                            preferred_element_type=jnp.float32)
        y = y + b_ref[...]
        mean = jnp.mean(y, axis=-1, keepdims=True)
        var = jnp.mean((y - mean) ** 2, axis=-1, keepdims=True)
        o_ref[...] = ((y - mean) * lax.rsqrt(var + LN_EPS) * lnw_ref[...]
                      + lnb_ref[...])

    return pl.pallas_call(
        body,
        grid=grid,
        in_specs=[
            pl.BlockSpec((blk, D_FEAT), lambda i: (i, 0)),
            pl.BlockSpec((NC, blk, D_FEAT), lambda i: (0, i, 0)),
            pl.BlockSpec((D_FEAT, D_FEAT), lambda i: (0, 0)),
            pl.BlockSpec((1, D_FEAT), lambda i: (0, 0)),
            pl.BlockSpec((1, D_FEAT), lambda i: (0, 0)),
            pl.BlockSpec((1, D_FEAT), lambda i: (0, 0)),
        ],
        out_specs=pl.BlockSpec((blk, D_FEAT), lambda i: (i, 0)),
        out_shape=jax.ShapeDtypeStruct((N_NODES, D_FEAT), jnp.float32),
    )(x, partials, w, b, ln_w, ln_b)


def kernel(x, edge_index, batch, W, b, ln_w, ln_b):
    src = edge_index[0]
    dst = edge_index[1]
    partials = _sc_aggregate(x, src, dst)
    return _tc_update(x, partials, W, b.reshape(1, D_FEAT),
                      ln_w.reshape(1, D_FEAT), ln_b.reshape(1, D_FEAT))


# speedup vs baseline: 11.9799x; 1.8109x over previous
import functools

import jax
import jax.numpy as jnp
from jax import lax
from jax.experimental import pallas as pl
from jax.experimental.pallas import tpu as pltpu
from jax.experimental.pallas import tpu_sc as plsc

N_NODES = 10000
D_FEAT = 128
LN_EPS = 1e-5

NC = 2
NS = 16
CH = 40
RING = 5
PF = 3
LAG = RING - PF
ZROWS = 16
NPAD = 10240


def _sc_aggregate(x, src, dst):
    e = src.shape[0]
    epw = e // (NC * NS)
    nch = epw // CH
    ngrp = nch // RING
    assert nch == ngrp * RING and epw == nch * CH
    rows_per_sub = NPAD // NS

    dst4 = dst.reshape(NC * NS, ngrp, RING, CH)
    mesh = plsc.VectorSubcoreMesh(core_axis_name="c", subcore_axis_name="s")

    scratch = (
        [pltpu.VMEM_SHARED((NPAD, D_FEAT), jnp.float32)]
        + [pltpu.VMEM((epw,), jnp.int32)]
        + [pltpu.VMEM((2 * RING, CH), jnp.int32)]
        + [pltpu.VMEM((CH, D_FEAT), jnp.float32)] * RING
        + [pltpu.VMEM((ZROWS, D_FEAT), jnp.float32)]
        + [pltpu.SemaphoreType.DMA] * (2 * RING + 1)
    )

    @functools.partial(
        pl.kernel,
        out_type=jax.ShapeDtypeStruct((NC, NPAD, D_FEAT), jnp.float32),
        mesh=mesh,
        scratch_types=scratch,
    )
    def k(x_hbm, src_hbm, dst4_hbm, out_hbm, *scr):
        aggr_sh, src_blk, dst_blk = scr[0], scr[1], scr[2]
        rowbufs = scr[3:3 + RING]
        zbuf = scr[3 + RING]
        gsems = scr[4 + RING:4 + 2 * RING]
        ssems = scr[4 + 2 * RING:4 + 3 * RING]
        isem = scr[4 + 3 * RING]

        c_ax = lax.axis_index("c")
        s_ax = lax.axis_index("s")
        w = c_ax * NS + s_ax

        def issue_gather(c, r):
            pltpu.async_copy(x_hbm.at[src_blk.at[pl.ds(c * CH, CH)]],
                             rowbufs[r], gsems[r])

        def wait_gather(c, r):
            pltpu.make_async_copy(x_hbm.at[src_blk.at[pl.ds(c * CH, CH)]],
                                  rowbufs[r], gsems[r]).wait()

        def issue_scatter(row, r):
            pltpu.async_copy(rowbufs[r], aggr_sh.at[dst_blk.at[row]],
                             ssems[r], add=True)

        def wait_scatter(row, r):
            pltpu.make_async_copy(rowbufs[r], aggr_sh.at[dst_blk.at[row]],
                                  ssems[r]).wait()

        def issue_idx(g, poff):
            pltpu.async_copy(dst4_hbm.at[w, g],
                             dst_blk.at[pl.ds(poff, RING)], isem)

        def wait_idx(g, poff):
            pltpu.make_async_copy(dst4_hbm.at[w, g],
                                  dst_blk.at[pl.ds(poff, RING)], isem).wait()

        pltpu.sync_copy(src_hbm.at[pl.ds(w * epw, epw)], src_blk)
        pltpu.sync_copy(dst4_hbm.at[w, 0], dst_blk.at[pl.ds(0, RING)])
        issue_idx(1, RING)

        for c0 in range(PF):
            issue_gather(c0, c0)

        @pl.loop(0, ZROWS)
        def _(i):
            @pl.loop(0, D_FEAT, step=16)
            def _(j):
                zbuf[i, pl.ds(j, 16)] = jnp.zeros((16,), jnp.float32)

        @pl.loop(0, rows_per_sub // ZROWS)
        def _(i):
            pltpu.sync_copy(
                zbuf, aggr_sh.at[pl.ds(s_ax * rows_per_sub + i * ZROWS,
                                       ZROWS)])

        plsc.subcore_barrier()

        def step(c, r, poff, do_wait_s, do_issue_g):
            wait_gather(c, r)
            issue_scatter(poff + r, r)
            if do_wait_s:
                drow = (poff + r - LAG) if r >= LAG else (
                    (RING - poff) + r - LAG + RING)
                wait_scatter(drow, (r - LAG) % RING)
            if do_issue_g:
                issue_gather(c + PF, (r + PF) % RING)

        for r in range(RING):
            step(r, r, 0, do_wait_s=(r >= LAG), do_issue_g=True)

        @pl.loop(0, ngrp - 2)
        def _(q):
            g = q + 1
            poff = (g % 2) * RING
            wait_idx(g, poff)
            for r in range(RING):
                step(g * RING + r, r, poff, True, True)
                if r == LAG - 1:
                    issue_idx(g + 1, RING - poff)

        gl = ngrp - 1
        wait_idx(gl, RING)
        for r in range(RING):
            c = gl * RING + r
            step(c, r, RING, do_wait_s=True, do_issue_g=(c + PF < nch))
        for c in range(nch - LAG, nch):
            wait_scatter(RING + (c % RING), c % RING)

        plsc.subcore_barrier()

        pltpu.sync_copy(
            aggr_sh.at[pl.ds(s_ax * rows_per_sub, rows_per_sub)],
            out_hbm.at[c_ax, pl.ds(s_ax * rows_per_sub, rows_per_sub)])

    return k(x, src, dst4)


def _tc_update(x, partials, w, b, ln_w, ln_b):
    blk = 1000
    grid = (N_NODES // blk,)
    partials = lax.slice(partials, (0, 0, 0), (NC, N_NODES, D_FEAT))

    def body(x_ref, p_ref, w_ref, b_ref, lnw_ref, lnb_ref, o_ref):
        h = x_ref[...] + p_ref[0] + p_ref[1]
        y = lax.dot_general(h, w_ref[...], (((1,), (1,)), ((), ())),
                            preferred_element_type=jnp.float32)
        y = y + b_ref[...]
        mean = jnp.mean(y, axis=-1, keepdims=True)
        var = jnp.mean((y - mean) ** 2, axis=-1, keepdims=True)
        o_ref[...] = ((y - mean) * lax.rsqrt(var + LN_EPS) * lnw_ref[...]
                      + lnb_ref[...])

    return pl.pallas_call(
        body,
        grid=grid,
        in_specs=[
            pl.BlockSpec((blk, D_FEAT), lambda i: (i, 0)),
            pl.BlockSpec((NC, blk, D_FEAT), lambda i: (0, i, 0)),
            pl.BlockSpec((D_FEAT, D_FEAT), lambda i: (0, 0)),
            pl.BlockSpec((1, D_FEAT), lambda i: (0, 0)),
            pl.BlockSpec((1, D_FEAT), lambda i: (0, 0)),
            pl.BlockSpec((1, D_FEAT), lambda i: (0, 0)),
        ],
        out_specs=pl.BlockSpec((blk, D_FEAT), lambda i: (i, 0)),
        out_shape=jax.ShapeDtypeStruct((N_NODES, D_FEAT), jnp.float32),
    )(x, partials, w, b, ln_w, ln_b)


def kernel(x, edge_index, batch, W, b, ln_w, ln_b):
    src = edge_index[0]
    dst = edge_index[1]
    partials = _sc_aggregate(x, src, dst)
    return _tc_update(x, partials, W, b.reshape(1, D_FEAT),
                      ln_w.reshape(1, D_FEAT), ln_b.reshape(1, D_FEAT))


# speedup vs baseline: 12.6886x; 1.0592x over previous
import functools

import jax
import jax.numpy as jnp
from jax import lax
from jax.experimental import pallas as pl
from jax.experimental.pallas import tpu as pltpu
from jax.experimental.pallas import tpu_sc as plsc

N_NODES = 10000
D_FEAT = 128
LN_EPS = 1e-5

NC = 2
NS = 16
CH = 40
RING = 5
PF = 3
LAG = RING - PF
ZROWS = 16
NPAD = 10240
AGG_DT = jnp.float32


def _sc_aggregate(x, src, dst):
    e = src.shape[0]
    epw = e // (NC * NS)
    nch = epw // CH
    ngrp = nch // RING
    assert nch == ngrp * RING and epw == nch * CH
    rows_per_sub = NPAD // NS

    dst4 = dst.reshape(NC * NS, ngrp, RING, CH)
    mesh = plsc.VectorSubcoreMesh(core_axis_name="c", subcore_axis_name="s")

    scratch = (
        [pltpu.VMEM_SHARED((NPAD, D_FEAT), AGG_DT)]
        + [pltpu.VMEM((epw,), jnp.int32)]
        + [pltpu.VMEM((2 * RING, CH), jnp.int32)]
        + [pltpu.VMEM((CH, D_FEAT), AGG_DT)] * RING
        + [pltpu.VMEM((ZROWS, D_FEAT), AGG_DT)]
        + [pltpu.SemaphoreType.DMA] * (2 * RING + 1)
    )

    @functools.partial(
        pl.kernel,
        out_type=jax.ShapeDtypeStruct((NC * NS, NPAD // NS, D_FEAT), AGG_DT),
        mesh=mesh,
        scratch_types=scratch,
    )
    def k(x_hbm, src_hbm, dst4_hbm, out_hbm, *scr):
        aggr_sh, src_blk, dst_blk = scr[0], scr[1], scr[2]
        rowbufs = scr[3:3 + RING]
        zbuf = scr[3 + RING]
        gsems = scr[4 + RING:4 + 2 * RING]
        ssems = scr[4 + 2 * RING:4 + 3 * RING]
        isem = scr[4 + 3 * RING]

        c_ax = lax.axis_index("c")
        s_ax = lax.axis_index("s")
        w = c_ax * NS + s_ax

        def issue_gather(c, r):
            pltpu.async_copy(x_hbm.at[src_blk.at[pl.ds(c * CH, CH)]],
                             rowbufs[r], gsems[r])

        def wait_gather(c, r):
            pltpu.make_async_copy(x_hbm.at[src_blk.at[pl.ds(c * CH, CH)]],
                                  rowbufs[r], gsems[r]).wait()

        def issue_scatter(row, r):
            pltpu.async_copy(rowbufs[r], aggr_sh.at[dst_blk.at[row]],
                             ssems[r], add=True)

        def wait_scatter(row, r):
            pltpu.make_async_copy(rowbufs[r], aggr_sh.at[dst_blk.at[row]],
                                  ssems[r]).wait()

        def issue_idx(g, poff):
            pltpu.async_copy(dst4_hbm.at[w, g],
                             dst_blk.at[pl.ds(poff, RING)], isem)

        def wait_idx(g, poff):
            pltpu.make_async_copy(dst4_hbm.at[w, g],
                                  dst_blk.at[pl.ds(poff, RING)], isem).wait()

        pltpu.sync_copy(src_hbm.at[pl.ds(w * epw, epw)], src_blk)
        pltpu.sync_copy(dst4_hbm.at[w, 0], dst_blk.at[pl.ds(0, RING)])
        issue_idx(1, RING)

        for c0 in range(PF):
            issue_gather(c0, c0)

        zwidth = 32 if AGG_DT == jnp.bfloat16 else 16

        @pl.loop(0, ZROWS)
        def _(i):
            @pl.loop(0, D_FEAT, step=zwidth)
            def _(j):
                zbuf[i, pl.ds(j, zwidth)] = jnp.zeros((zwidth,), AGG_DT)

        @pl.loop(0, rows_per_sub // ZROWS)
        def _(i):
            zoff = pl.multiple_of(s_ax * rows_per_sub + i * ZROWS, ZROWS)
            pltpu.sync_copy(zbuf, aggr_sh.at[pl.ds(zoff, ZROWS)])

        plsc.subcore_barrier()

        def step(c, r, poff, do_wait_s, do_issue_g):
            wait_gather(c, r)
            issue_scatter(poff + r, r)
            if do_wait_s:
                drow = (poff + r - LAG) if r >= LAG else (
                    (RING - poff) + r - LAG + RING)
                wait_scatter(drow, (r - LAG) % RING)
            if do_issue_g:
                issue_gather(c + PF, (r + PF) % RING)

        for r in range(RING):
            step(r, r, 0, do_wait_s=(r >= LAG), do_issue_g=True)

        @pl.loop(0, ngrp - 2)
        def _(q):
            g = q + 1
            poff = (g % 2) * RING
            wait_idx(g, poff)
            for r in range(RING):
                step(g * RING + r, r, poff, True, True)
                if r == LAG - 1:
                    issue_idx(g + 1, RING - poff)

        gl = ngrp - 1
        lpoff = (gl % 2) * RING
        wait_idx(gl, lpoff)
        for r in range(RING):
            c = gl * RING + r
            step(c, r, lpoff, do_wait_s=True, do_issue_g=(c + PF < nch))
        for c in range(nch - LAG, nch):
            wait_scatter(lpoff + (c % RING), c % RING)

        plsc.subcore_barrier()

        out_off = pl.multiple_of(s_ax * rows_per_sub, rows_per_sub)
        pltpu.sync_copy(
            aggr_sh.at[pl.ds(out_off, rows_per_sub)],
            out_hbm.at[w])

    return k(x, src, dst4).reshape(NC, NPAD, D_FEAT)


def _tc_update(x, partials, w, b, ln_w, ln_b):
    blk = 2000
    grid = (N_NODES // blk,)

    def body(x_ref, p_ref, w_ref, b_ref, lnw_ref, lnb_ref, o_ref):
        h = (x_ref[...] + p_ref[0].astype(jnp.float32)
             + p_ref[1].astype(jnp.float32))
        y = lax.dot_general(h, w_ref[...], (((1,), (1,)), ((), ())),
                            preferred_element_type=jnp.float32)
        y = y + b_ref[...]
        mean = jnp.mean(y, axis=-1, keepdims=True)
        var = jnp.mean((y - mean) ** 2, axis=-1, keepdims=True)
        o_ref[...] = ((y - mean) * lax.rsqrt(var + LN_EPS) * lnw_ref[...]
                      + lnb_ref[...])

    return pl.pallas_call(
        body,
        grid=grid,
        in_specs=[
            pl.BlockSpec((blk, D_FEAT), lambda i: (i, 0)),
            pl.BlockSpec((NC, blk, D_FEAT), lambda i: (0, i, 0)),
            pl.BlockSpec((D_FEAT, D_FEAT), lambda i: (0, 0)),
            pl.BlockSpec((1, D_FEAT), lambda i: (0, 0)),
            pl.BlockSpec((1, D_FEAT), lambda i: (0, 0)),
            pl.BlockSpec((1, D_FEAT), lambda i: (0, 0)),
        ],
        out_specs=pl.BlockSpec((blk, D_FEAT), lambda i: (i, 0)),
        out_shape=jax.ShapeDtypeStruct((N_NODES, D_FEAT), jnp.float32),
    )(x, partials, w, b, ln_w, ln_b)


def kernel(x, edge_index, batch, W, b, ln_w, ln_b):
    src = edge_index[0]
    dst = edge_index[1]
    partials = _sc_aggregate(x, src, dst)
    return _tc_update(x, partials, W, b.reshape(1, D_FEAT),
                      ln_w.reshape(1, D_FEAT), ln_b.reshape(1, D_FEAT))


# speedup vs baseline: 12.7305x; 1.0033x over previous
import functools

import jax
import jax.numpy as jnp
from jax import lax
from jax.experimental import pallas as pl
from jax.experimental.pallas import tpu as pltpu
from jax.experimental.pallas import tpu_sc as plsc

N_NODES = 10000
D_FEAT = 128
LN_EPS = 1e-5

NC = 2
NS = 16
CH = 40
RING = 5
PF = 3
LAG = RING - PF
ZROWS = 64
NPAD = 10240
AGG_DT = jnp.float32


def _sc_aggregate(x, edge_index):
    e = edge_index.shape[1]
    epw = e // (NC * NS)
    nch = epw // CH
    ngrp = nch // RING
    assert nch == ngrp * RING and epw == nch * CH
    rows_per_sub = NPAD // NS

    src1 = edge_index[0]
    dst4 = jnp.reshape(edge_index[1], (NC * NS, ngrp, RING, CH))
    mesh = plsc.VectorSubcoreMesh(core_axis_name="c", subcore_axis_name="s")

    scratch = (
        [pltpu.VMEM_SHARED((NPAD, D_FEAT), AGG_DT)]
        + [pltpu.VMEM((epw,), jnp.int32)]
        + [pltpu.VMEM((2 * RING, CH), jnp.int32)]
        + [pltpu.VMEM((CH, D_FEAT), AGG_DT)] * RING
        + [pltpu.VMEM((ZROWS, D_FEAT), AGG_DT)]
        + [pltpu.SemaphoreType.DMA] * (2 * RING + 1)
    )

    @functools.partial(
        pl.kernel,
        out_type=jax.ShapeDtypeStruct((NC * NS, NPAD // NS, D_FEAT), AGG_DT),
        mesh=mesh,
        scratch_types=scratch,
    )
    def k(x_hbm, src_hbm, dst4_hbm, out_hbm, *scr):
        aggr_sh, src_blk, dst_blk = scr[0], scr[1], scr[2]
        rowbufs = scr[3:3 + RING]
        zbuf = scr[3 + RING]
        gsems = scr[4 + RING:4 + 2 * RING]
        ssems = scr[4 + 2 * RING:4 + 3 * RING]
        isem = scr[4 + 3 * RING]

        c_ax = lax.axis_index("c")
        s_ax = lax.axis_index("s")
        w = c_ax * NS + s_ax

        def issue_gather(gg, rr, slot):
            idx = src_blk.at[pl.ds((gg * RING + rr) * CH, CH)]
            pltpu.async_copy(x_hbm.at[idx], rowbufs[slot], gsems[slot])

        def wait_gather(gg, rr, slot):
            idx = src_blk.at[pl.ds((gg * RING + rr) * CH, CH)]
            pltpu.make_async_copy(x_hbm.at[idx], rowbufs[slot],
                                  gsems[slot]).wait()

        def issue_scatter(row, r):
            pltpu.async_copy(rowbufs[r], aggr_sh.at[dst_blk.at[row]],
                             ssems[r], add=True)

        def wait_scatter(row, r):
            pltpu.make_async_copy(rowbufs[r], aggr_sh.at[dst_blk.at[row]],
                                  ssems[r]).wait()

        def issue_idx(g, poff):
            pltpu.async_copy(dst4_hbm.at[w, g],
                             dst_blk.at[pl.ds(poff, RING)], isem)

        def wait_idx(g, poff):
            pltpu.make_async_copy(dst4_hbm.at[w, g],
                                  dst_blk.at[pl.ds(poff, RING)], isem).wait()

        pltpu.sync_copy(src_hbm.at[pl.ds(w * epw, epw)], src_blk)
        pltpu.sync_copy(dst4_hbm.at[w, 0], dst_blk.at[pl.ds(0, RING)])
        issue_idx(1, RING)

        for c0 in range(PF):
            issue_gather(0, c0, c0)

        zwidth = 32 if AGG_DT == jnp.bfloat16 else 16

        @pl.loop(0, ZROWS)
        def _(i):
            @pl.loop(0, D_FEAT, step=zwidth)
            def _(j):
                zbuf[i, pl.ds(j, zwidth)] = jnp.zeros((zwidth,), AGG_DT)

        for j in range(NS):
            @pl.when(s_ax == j)
            def _():
                for i in range(rows_per_sub // ZROWS):
                    pltpu.sync_copy(
                        zbuf,
                        aggr_sh.at[pl.ds(j * rows_per_sub + i * ZROWS,
                                         ZROWS)])

        plsc.subcore_barrier()

        def step(g, r, poff, do_wait_s, do_issue_g):
            wait_gather(g, r, r)
            issue_scatter(poff + r, r)
            if do_wait_s:
                drow = (poff + r - LAG) if r >= LAG else (
                    (RING - poff) + r - LAG + RING)
                wait_scatter(drow, (r - LAG) % RING)
            if do_issue_g:
                if r + PF < RING:
                    issue_gather(g, r + PF, r + PF)
                else:
                    issue_gather(g + 1, r + PF - RING, r + PF - RING)

        for r in range(RING):
            step(0, r, 0, do_wait_s=(r >= LAG), do_issue_g=True)

        @pl.loop(0, ngrp - 2)
        def _(q):
            g = q + 1
            poff = (g % 2) * RING
            wait_idx(g, poff)
            for r in range(RING):
                step(g, r, poff, True, True)
                if r == LAG - 1:
                    issue_idx(g + 1, RING - poff)

        gl = ngrp - 1
        lpoff = (gl % 2) * RING
        wait_idx(gl, lpoff)
        for r in range(RING):
            step(gl, r, lpoff, do_wait_s=True,
                 do_issue_g=(gl * RING + r + PF < nch))
        for c in range(nch - LAG, nch):
            wait_scatter(lpoff + (c % RING), c % RING)

        plsc.subcore_barrier()

        for j in range(NS):
            @pl.when(s_ax == j)
            def _():
                pltpu.sync_copy(
                    aggr_sh.at[pl.ds(j * rows_per_sub, rows_per_sub)],
                    out_hbm.at[w])

    return k(x, src1, dst4).reshape(NC, NPAD, D_FEAT)


def _tc_update(x, partials, w, b, ln_w, ln_b):
    blk = 2000
    grid = (N_NODES // blk,)

    def body(x_ref, p_ref, w_ref, b_ref, lnw_ref, lnb_ref, o_ref):
        h = (x_ref[...] + p_ref[0].astype(jnp.float32)
             + p_ref[1].astype(jnp.float32))
        y = lax.dot_general(h, w_ref[...], (((1,), (1,)), ((), ())),
                            preferred_element_type=jnp.float32)
        y = y + b_ref[...]
        mean = jnp.mean(y, axis=-1, keepdims=True)
        var = jnp.mean((y - mean) ** 2, axis=-1, keepdims=True)
        o_ref[...] = ((y - mean) * lax.rsqrt(var + LN_EPS) * lnw_ref[...]
                      + lnb_ref[...])

    return pl.pallas_call(
        body,
        grid=grid,
        in_specs=[
            pl.BlockSpec((blk, D_FEAT), lambda i: (i, 0)),
            pl.BlockSpec((NC, blk, D_FEAT), lambda i: (0, i, 0)),
            pl.BlockSpec((D_FEAT, D_FEAT), lambda i: (0, 0)),
            pl.BlockSpec((1, D_FEAT), lambda i: (0, 0)),
            pl.BlockSpec((1, D_FEAT), lambda i: (0, 0)),
            pl.BlockSpec((1, D_FEAT), lambda i: (0, 0)),
        ],
        out_specs=pl.BlockSpec((blk, D_FEAT), lambda i: (i, 0)),
        out_shape=jax.ShapeDtypeStruct((N_NODES, D_FEAT), jnp.float32),
    )(x, partials, w, b, ln_w, ln_b)


def kernel(x, edge_index, batch, W, b, ln_w, ln_b):
    partials = _sc_aggregate(x, edge_index)
    return _tc_update(x, partials, W, b.reshape(1, D_FEAT),
                      ln_w.reshape(1, D_FEAT), ln_b.reshape(1, D_FEAT))
